# baseline (speedup 1.0000x reference)
import jax
import jax.numpy as jnp
import numpy as np
from jax import lax
from jax.experimental import pallas as pl
from jax.experimental.pallas import tpu as pltpu
from jax.experimental.pallas import tpu_sc as plsc

N = 10000
E = 320000
DIM = 128
H = 256
G = 128
TEMP = 0.66
IMIN = -0.1
IMAX = 1.1
L0_SHIFT = float(TEMP * np.log2(-IMIN / IMAX))

NC = 2
NS = 16
NW = NC * NS
NH = 2
E2 = E // NH
CE = E2 // NW
CH = 40
NCHUNK = CE // CH

EB = 640
NEB = E2 // EB

NP = 10240
DUMP = N
NB = 640
NBLK = NP // NB


XCH = 80
NXCH = N // XCH


def _xe_body(femb, xidx, xe_out, iv, rv, sem):
    c = lax.axis_index("c")
    s = lax.axis_index("s")
    wid = c * NS + s

    for k in range((NXCH + NW - 1) // NW):
        cid = wid + k * NW

        @pl.when(cid < NXCH)
        def _():
            base = cid * XCH
            pltpu.sync_copy(xidx.at[pl.ds(base, XCH)], iv)
            pltpu.async_copy(femb.at[iv], rv, sem).wait()
            pltpu.sync_copy(rv, xe_out.at[pl.ds(base, XCH)])


def _sc_xe(femb, xidx):
    mesh = plsc.VectorSubcoreMesh(core_axis_name="c", subcore_axis_name="s",
                                  num_cores=NC, num_subcores=NS)
    f = pl.kernel(
        _xe_body,
        out_type=jax.ShapeDtypeStruct((N, DIM), jnp.float32),
        mesh=mesh,
        scratch_types=(
            pltpu.VMEM((XCH,), jnp.int32),
            pltpu.VMEM((XCH, DIM), jnp.float32),
            pltpu.SemaphoreType.DMA,
        ),
    )
    return f(femb, xidx)


def _gather_body(xe, fedge, src, dst, ea0, ea1, pe_out, ge_out,
                 is0, id0, ie0, if0, is1, id1, ie1, if1,
                 ra0, rb0, rc0, rd0, ra1, rb1, rc1, rd1,
                 os1, os2, gsem0, gsem1, isem0, isem1, wsem):
    c = lax.axis_index("c")
    s = lax.axis_index("s")
    wid = c * NS + s
    iall = ((is0, id0, ie0, if0), (is1, id1, ie1, if1))
    srcs = (src, dst, ea0, ea1)
    bufs = ((ra0, rb0, rc0, rd0), (ra1, rb1, rc1, rd1))
    gsem = (gsem0, gsem1)
    isem = (isem0, isem1)

    def fire_idx(ci, p):
        base = wid * CE + ci * CH
        for k in range(4):
            pltpu.async_copy(srcs[k].at[pl.ds(base, CH)], iall[p][k],
                             isem[p])

    def fire_gathers(ci, p):
        ra, rb, rc, rd = bufs[p]
        for k in range(4):
            pltpu.make_async_copy(srcs[k].at[pl.ds(0, CH)], iall[p][k],
                                  isem[p]).wait()
        pltpu.async_copy(xe.at[iall[p][0]], ra, gsem[p])
        pltpu.async_copy(xe.at[iall[p][1]], rb, gsem[p])
        pltpu.async_copy(fedge.at[iall[p][2]], rc, gsem[p])
        pltpu.async_copy(fedge.at[iall[p][3]], rd, gsem[p])

    def step(ci, p, first, last):
        ra, rb, rc, rd = bufs[p]
        for buf in (ra, rb, rc, rd):
            pltpu.make_async_copy(xe.at[pl.ds(0, CH)], buf, gsem[p]).wait()
        if not last:
            fire_idx(ci + 2, p)
        if not first:
            pltpu.make_async_copy(pe_out.at[pl.ds(0, CH)], os1, wsem).wait()
            pltpu.make_async_copy(ge_out.at[pl.ds(0, CH)], os2, wsem).wait()

        def mul2(r, _):
            for cc in range(DIM // 16):
                sl = pl.ds(cc * 16, 16)
                os1[r, sl] = ra[r, sl] * rb[r, sl]
                os2[r, sl] = rc[r, sl] * rd[r, sl]
            return 0

        lax.fori_loop(0, CH, mul2, 0)
        base = wid * CE + ci * CH
        pltpu.async_copy(os1, pe_out.at[pl.ds(base, CH)], wsem)
        pltpu.async_copy(os2, ge_out.at[pl.ds(base, CH)], wsem)
        if not last:
            fire_gathers(ci + 2, p)

    fire_idx(0, 0)
    fire_gathers(0, 0)
    fire_idx(1, 1)
    fire_gathers(1, 1)
    step(0, 0, first=True, last=False)

    def loop(it, _):
        step(2 * it + 1, 1, first=False, last=False)
        step(2 * it + 2, 0, first=False, last=False)
        return 0

    lax.fori_loop(0, (NCHUNK - 3) // 2, loop, 0)
    step(NCHUNK - 2, 1, first=False, last=True)
    step(NCHUNK - 1, 0, first=False, last=True)
    pltpu.make_async_copy(pe_out.at[pl.ds(0, CH)], os1, wsem).wait()
    pltpu.make_async_copy(ge_out.at[pl.ds(0, CH)], os2, wsem).wait()


def _sc_gather(xe, fedge, src, dst, ea0, ea1):
    mesh = plsc.VectorSubcoreMesh(core_axis_name="c", subcore_axis_name="s",
                                  num_cores=NC, num_subcores=NS)
    f = pl.kernel(
        _gather_body,
        out_type=(jax.ShapeDtypeStruct((E2, DIM), jnp.float32),
                  jax.ShapeDtypeStruct((E2, DIM), jnp.float32)),
        mesh=mesh,
        scratch_types=(
            pltpu.VMEM((CH,), jnp.int32),
            pltpu.VMEM((CH,), jnp.int32),
            pltpu.VMEM((CH,), jnp.int32),
            pltpu.VMEM((CH,), jnp.int32),
            pltpu.VMEM((CH,), jnp.int32),
            pltpu.VMEM((CH,), jnp.int32),
            pltpu.VMEM((CH,), jnp.int32),
            pltpu.VMEM((CH,), jnp.int32),
            pltpu.VMEM((CH, DIM), jnp.float32),
            pltpu.VMEM((CH, DIM), jnp.float32),
            pltpu.VMEM((CH, DIM), jnp.float32),
            pltpu.VMEM((CH, DIM), jnp.float32),
            pltpu.VMEM((CH, DIM), jnp.float32),
            pltpu.VMEM((CH, DIM), jnp.float32),
            pltpu.VMEM((CH, DIM), jnp.float32),
            pltpu.VMEM((CH, DIM), jnp.float32),
            pltpu.VMEM((CH, DIM), jnp.float32),
            pltpu.VMEM((CH, DIM), jnp.float32),
            pltpu.SemaphoreType.DMA,
            pltpu.SemaphoreType.DMA,
            pltpu.SemaphoreType.DMA,
            pltpu.SemaphoreType.DMA,
            pltpu.SemaphoreType.DMA,
        ),
    )
    return f(xe, fedge, src, dst, ea0, ea1)


def _mlp_body(pe_ref, ge_ref, dst_ref, w1, b1, w2, b2, l1w, l1b, l2w, l2b,
              msg_ref, mdst_ref, l0_ref, ne_ref):
    i = pl.program_id(0)
    ge = ge_ref[...]
    h = jnp.maximum(
        lax.dot_general(ge, w1[...], (((1,), (1,)), ((), ())),
                        preferred_element_type=jnp.float32) + b1[...], 0.0)
    loc = jnp.sum(h * w2[...], axis=1, keepdims=True) + b2[...]
    sig = jax.nn.sigmoid(loc)
    sgate = jnp.clip(sig * (IMAX - IMIN) + IMIN, 0.0, 1.0)
    maskv = (sgate > 0.0).astype(jnp.float32)
    l0p = jnp.sum(jax.nn.sigmoid(loc - L0_SHIFT))
    nep = jnp.sum(maskv)

    pe = pe_ref[...]
    h2 = jnp.maximum(
        lax.dot_general(pe, l1w[...], (((1,), (1,)), ((), ())),
                        preferred_element_type=jnp.float32) + l1b[...], 0.0)
    pw = lax.dot_general(h2, l2w[...], (((1,), (1,)), ((), ())),
                         preferred_element_type=jnp.float32) + l2b[...]
    msg_ref[...] = pw * sgate

    dstv = dst_ref[0]
    mdst_ref[0] = jnp.where(sgate > 0.0, dstv, DUMP)

    @pl.when(i == 0)
    def _():
        l0_ref[...] = jnp.zeros((1, 1), jnp.float32)
        ne_ref[...] = jnp.zeros((1, 1), jnp.float32)

    l0_ref[...] += jnp.reshape(l0p, (1, 1))
    ne_ref[...] += jnp.reshape(nep, (1, 1))


def _tc_mlp(pe, ge, dst3, w1, b1, w2, b2, l1w, l1b, l2w, l2b):
    full = lambda shp: pl.BlockSpec(shp, lambda i: (0,) * len(shp))
    return pl.pallas_call(
        _mlp_body,
        grid=(NEB,),
        in_specs=[
            pl.BlockSpec((EB, DIM), lambda i: (i, 0)),
            pl.BlockSpec((EB, DIM), lambda i: (i, 0)),
            pl.BlockSpec((1, EB, 1), lambda i: (i, 0, 0)),
            full((H, DIM)), full((1, H)), full((1, H)), full((1, 1)),
            full((H, DIM)), full((1, H)), full((DIM, H)), full((1, DIM)),
        ],
        out_specs=[
            pl.BlockSpec((EB, DIM), lambda i: (i, 0)),
            pl.BlockSpec((1, EB, 1), lambda i: (i, 0, 0)),
            pl.BlockSpec((1, 1), lambda i: (0, 0)),
            pl.BlockSpec((1, 1), lambda i: (0, 0)),
        ],
        out_shape=[
            jax.ShapeDtypeStruct((E2, DIM), jnp.float32),
            jax.ShapeDtypeStruct((NEB, EB, 1), jnp.int32),
            jax.ShapeDtypeStruct((1, 1), jnp.float32),
            jax.ShapeDtypeStruct((1, 1), jnp.float32),
        ],
    )(pe, ge, dst3, w1, b1, w2, b2, l1w, l1b, l2w, l2b)


NPS = NP // NS
ZR = 128
SCH = 80


NSCH = CE // SCH


def _scatter_msg_body(msgx, dste, msum, mv0, mv1, dv0, dv1, zb,
                      lsem0, lsem1, ssem0, ssem1, acc):
    c = lax.axis_index("c")
    s = lax.axis_index("s")
    mv = (mv0, mv1)
    dv = (dv0, dv1)
    lsem = (lsem0, lsem1)
    ssem = (ssem0, ssem1)

    def zrow(r, _):
        for cc in range(DIM // 16):
            zb[r, pl.ds(cc * 16, 16)] = jnp.zeros((16,), jnp.float32)
        return 0

    lax.fori_loop(0, ZR, zrow, 0)
    stripe = s * NPS
    for k in range(NPS // ZR):
        pltpu.sync_copy(zb, acc.at[pl.ds(stripe + k * ZR, ZR)])
    plsc.subcore_barrier()

    ebase = c * (E2 // NC) + s * CE

    def fire_loads(ci, p):
        pltpu.async_copy(dste.at[pl.ds(ebase + ci * SCH, SCH)], dv[p],
                         lsem[p])
        pltpu.async_copy(msgx.at[pl.ds(ebase + ci * SCH, SCH)], mv[p],
                         lsem[p])

    def step(ci, p, first, last):
        q = 1 - p
        pltpu.make_async_copy(dste.at[pl.ds(0, SCH)], dv[p], lsem[p]).wait()
        pltpu.make_async_copy(msgx.at[pl.ds(0, SCH)], mv[p], lsem[p]).wait()
        pltpu.async_copy(mv[p], acc.at[dv[p]], ssem[p], add=True)
        if not first:
            pltpu.make_async_copy(msgx.at[pl.ds(0, SCH)], mv[q],
                                  ssem[q]).wait()
        if not last:
            fire_loads(ci + 1, q)

    fire_loads(0, 0)
    step(0, 0, first=True, last=False)

    def loop(it, _):
        step(2 * it + 1, 1, first=False, last=False)
        step(2 * it + 2, 0, first=False, last=False)
        return 0

    lax.fori_loop(0, (NSCH - 3) // 2, loop, 0)
    step(NSCH - 2, 1, first=False, last=False)
    step(NSCH - 1, 0, first=False, last=True)
    pltpu.make_async_copy(msgx.at[pl.ds(0, SCH)], mv0, ssem0).wait()
    plsc.subcore_barrier()

    for k in range(NPS // ZR):
        sl = pl.ds(stripe + k * ZR, ZR)
        pltpu.sync_copy(acc.at[sl], zb)
        pltpu.sync_copy(zb, msum.at[pl.ds(c * NP + stripe + k * ZR, ZR)])


def _sc_scatter_msg(msgx, dste):
    mesh = plsc.VectorSubcoreMesh(core_axis_name="c", subcore_axis_name="s",
                                  num_cores=NC, num_subcores=NS)
    f = pl.kernel(
        _scatter_msg_body,
        out_type=jax.ShapeDtypeStruct((NC * NP, DIM), jnp.float32),
        mesh=mesh,
        scratch_types=(
            pltpu.VMEM((SCH, DIM), jnp.float32),
            pltpu.VMEM((SCH, DIM), jnp.float32),
            pltpu.VMEM((SCH,), jnp.int32),
            pltpu.VMEM((SCH,), jnp.int32),
            pltpu.VMEM((ZR, DIM), jnp.float32),
            pltpu.SemaphoreType.DMA,
            pltpu.SemaphoreType.DMA,
            pltpu.SemaphoreType.DMA,
            pltpu.SemaphoreType.DMA,
            pltpu.VMEM_SHARED((NP, DIM), jnp.float32),
        ),
    )
    return f(msgx, dste)


def _scatter_cnt_body(mdst, csum, ones, dv0, dv1, zb,
                      lsem0, lsem1, ssem0, ssem1, acc):
    c = lax.axis_index("c")
    s = lax.axis_index("s")
    dv = (dv0, dv1)
    lsem = (lsem0, lsem1)
    ssem = (ssem0, ssem1)

    def zrow(r, _):
        for cc in range(DIM // 16):
            zb[r, pl.ds(cc * 16, 16)] = jnp.zeros((16,), jnp.float32)
        return 0

    lax.fori_loop(0, ZR, zrow, 0)

    e0 = jnp.where(lax.iota(jnp.int32, 16) == 0,
                   jnp.float32(1.0), jnp.float32(0.0))

    def orow(r, _):
        ones[r, pl.ds(0, 16)] = e0
        for cc in range(1, DIM // 16):
            ones[r, pl.ds(cc * 16, 16)] = jnp.zeros((16,), jnp.float32)
        return 0

    lax.fori_loop(0, SCH, orow, 0)
    stripe = s * NPS
    for k in range(NPS // ZR):
        pltpu.sync_copy(zb, acc.at[pl.ds(stripe + k * ZR, ZR)])
    plsc.subcore_barrier()

    ebase = c * (E2 // NC) + s * CE

    def fire_load(ci, p):
        pltpu.async_copy(mdst.at[pl.ds(ebase + ci * SCH, SCH)], dv[p],
                         lsem[p])

    def step(ci, p, first, last):
        q = 1 - p
        pltpu.make_async_copy(mdst.at[pl.ds(0, SCH)], dv[p], lsem[p]).wait()
        pltpu.async_copy(ones, acc.at[dv[p]], ssem[p], add=True)
        if not first:
            pltpu.make_async_copy(csum.at[pl.ds(0, SCH)], ones,
                                  ssem[q]).wait()
        if not last:
            fire_load(ci + 1, q)

    fire_load(0, 0)
    step(0, 0, first=True, last=False)

    def loop(it, _):
        step(2 * it + 1, 1, first=False, last=False)
        step(2 * it + 2, 0, first=False, last=False)
        return 0

    lax.fori_loop(0, (NSCH - 3) // 2, loop, 0)
    step(NSCH - 2, 1, first=False, last=False)
    step(NSCH - 1, 0, first=False, last=True)
    pltpu.make_async_copy(csum.at[pl.ds(0, SCH)], ones, ssem0).wait()
    plsc.subcore_barrier()

    for k in range(NPS // ZR):
        sl = pl.ds(stripe + k * ZR, ZR)
        pltpu.sync_copy(acc.at[sl], zb)
        pltpu.sync_copy(zb, csum.at[pl.ds(c * NP + stripe + k * ZR, ZR)])


def _sc_scatter_cnt(mdst):
    mesh = plsc.VectorSubcoreMesh(core_axis_name="c", subcore_axis_name="s",
                                  num_cores=NC, num_subcores=NS)
    f = pl.kernel(
        _scatter_cnt_body,
        out_type=jax.ShapeDtypeStruct((NC * NP, DIM), jnp.float32),
        mesh=mesh,
        scratch_types=(
            pltpu.VMEM((SCH, DIM), jnp.float32),
            pltpu.VMEM((SCH,), jnp.int32),
            pltpu.VMEM((SCH,), jnp.int32),
            pltpu.VMEM((ZR, DIM), jnp.float32),
            pltpu.SemaphoreType.DMA,
            pltpu.SemaphoreType.DMA,
            pltpu.SemaphoreType.DMA,
            pltpu.SemaphoreType.DMA,
            pltpu.VMEM_SHARED((NP, DIM), jnp.float32),
        ),
    )
    return f(mdst)


def _fin_body(msum_ref, csum_ref, msum2_ref, csum2_ref, batch_ref, gw, gb,
              out_ref, l2_ref, pooled, gcnt):
    i = pl.program_id(0)

    @pl.when(i == 0)
    def _():
        pooled[...] = jnp.zeros((G, DIM), jnp.float32)
        gcnt[...] = jnp.zeros((G, 1), jnp.float32)
        l2_ref[...] = jnp.zeros((1, 1), jnp.float32)

    mm = msum_ref[...]
    mm2 = msum2_ref[...]
    m = mm[0] + mm[1] + mm2[0] + mm2[1]
    cc = csum_ref[...]
    cc2 = csum2_ref[...]
    cn = (cc[0, :, 0:1] + cc[1, :, 0:1]
          + cc2[0, :, 0:1] + cc2[1, :, 0:1])
    upd = m / jnp.maximum(cn, 1.0)
    l2_ref[...] += jnp.reshape(jnp.sum(upd * upd), (1, 1))

    bv = batch_ref[0, 0, :]
    oh = (bv[None, :] == lax.broadcasted_iota(jnp.int32, (G, NB), 0)
          ).astype(jnp.float32)
    pooled[...] += lax.dot_general(oh, upd, (((1,), (0,)), ((), ())),
                                   preferred_element_type=jnp.float32)
    gcnt[...] += jnp.sum(oh, axis=1, keepdims=True)

    @pl.when(i == NBLK - 1)
    def _():
        p = pooled[...] / jnp.maximum(gcnt[...], 1.0)
        out_ref[...] = lax.dot_general(p, gw[...], (((1,), (1,)), ((), ())),
                                       preferred_element_type=jnp.float32) \
            + gb[...]


def _tc_finish(msum, csum, msum2, csum2, batch3, gw, gb):
    return pl.pallas_call(
        _fin_body,
        grid=(NBLK,),
        in_specs=[
            pl.BlockSpec((2, NB, DIM), lambda i: (0, i, 0)),
            pl.BlockSpec((2, NB, DIM), lambda i: (0, i, 0)),
            pl.BlockSpec((2, NB, DIM), lambda i: (0, i, 0)),
            pl.BlockSpec((2, NB, DIM), lambda i: (0, i, 0)),
            pl.BlockSpec((1, 1, NB), lambda i: (i, 0, 0)),
            pl.BlockSpec((2, DIM), lambda i: (0, 0)),
            pl.BlockSpec((1, 2), lambda i: (0, 0)),
        ],
        out_specs=[
            pl.BlockSpec((G, 2), lambda i: (0, 0)),
            pl.BlockSpec((1, 1), lambda i: (0, 0)),
        ],
        out_shape=[
            jax.ShapeDtypeStruct((G, 2), jnp.float32),
            jax.ShapeDtypeStruct((1, 1), jnp.float32),
        ],
        scratch_shapes=[
            pltpu.VMEM((G, DIM), jnp.float32),
            pltpu.VMEM((G, 1), jnp.float32),
        ],
    )(msum, csum, msum2, csum2, batch3, gw, gb)


def kernel(x, edge_index, edge_attr, batch, is_training,
           feature_emb, fe_edge, lp_w1, lp_b1, lp_w2, lp_b2,
           lin1_w, lin1_b, lin2_w, lin2_b, g_w, g_b):
    xidx = x[:, 0].astype(jnp.int32)
    src = edge_index[0].astype(jnp.int32)
    dst = edge_index[1].astype(jnp.int32)
    ea0 = edge_attr[:, 0].astype(jnp.int32)
    ea1 = edge_attr[:, 1].astype(jnp.int32)

    xe = _sc_xe(feature_emb, xidx)

    msums, csums, l0s, nes = [], [], [], []
    for h in range(NH):
        sl = slice(h * E2, (h + 1) * E2)
        srch, dsth, ea0h, ea1h = src[sl], dst[sl], ea0[sl], ea1[sl]
        pe, ge = _sc_gather(xe, fe_edge, srch, dsth, ea0h, ea1h)
        msg, mdst3, l0_sum, ne_sum = _tc_mlp(
            pe, ge, dsth.reshape(NEB, EB, 1),
            lp_w1, lp_b1.reshape(1, H), lp_w2.reshape(1, H),
            lp_b2.reshape(1, 1),
            lin1_w, lin1_b.reshape(1, H), lin2_w, lin2_b.reshape(1, DIM))
        msums.append(_sc_scatter_msg(msg, dsth).reshape(NC, NP, DIM))
        csums.append(_sc_scatter_cnt(mdst3.reshape(E2)).reshape(NC, NP, DIM))
        l0s.append(l0_sum)
        nes.append(ne_sum)

    batch_pad = jnp.concatenate(
        [batch.astype(jnp.int32), jnp.full((NP - N,), G, jnp.int32)])
    out, l2 = _tc_finish(msums[0], csums[0], msums[1], csums[1],
                         batch_pad.reshape(NBLK, 1, NB),
                         g_w, g_b.reshape(1, 2))

    l0 = (l0s[0][0, 0] + l0s[1][0, 0]) / float(E)
    l2s = l2[0, 0]
    num_edges = (nes[0][0, 0] + nes[1][0, 0]).astype(jnp.int32)
    return out, l0, l2s, num_edges


# speedup vs baseline: 1.2646x; 1.2646x over previous
import jax
import jax.numpy as jnp
import numpy as np
from jax import lax
from jax.experimental import pallas as pl
from jax.experimental.pallas import tpu as pltpu
from jax.experimental.pallas import tpu_sc as plsc

N = 10000
E = 320000
DIM = 128
H = 256
G = 128
TEMP = 0.66
IMIN = -0.1
IMAX = 1.1
L0_SHIFT = float(TEMP * np.log2(-IMIN / IMAX))

NC = 2
NS = 16
NW = NC * NS
NH = 1
E2 = E // NH
CE = E2 // NW
CH = 80
NCHUNK = CE // CH

EB = 2560
NEB = E2 // EB

NP = 10240
DUMP = N
NB = 640
NBLK = NP // NB


XCH = 80
NXCH = N // XCH


def _xe_body(femb, xidx, xe_out, iv, rv, sem):
    c = lax.axis_index("c")
    s = lax.axis_index("s")
    wid = c * NS + s

    for k in range((NXCH + NW - 1) // NW):
        cid = wid + k * NW

        @pl.when(cid < NXCH)
        def _():
            base = cid * XCH
            pltpu.sync_copy(xidx.at[pl.ds(base, XCH)], iv)
            pltpu.async_copy(femb.at[iv], rv, sem).wait()
            pltpu.sync_copy(rv, xe_out.at[pl.ds(base, XCH)])


def _sc_xe(femb, xidx):
    mesh = plsc.VectorSubcoreMesh(core_axis_name="c", subcore_axis_name="s",
                                  num_cores=NC, num_subcores=NS)
    f = pl.kernel(
        _xe_body,
        out_type=jax.ShapeDtypeStruct((N, DIM), jnp.float32),
        mesh=mesh,
        scratch_types=(
            pltpu.VMEM((XCH,), jnp.int32),
            pltpu.VMEM((XCH, DIM), jnp.float32),
            pltpu.SemaphoreType.DMA,
        ),
    )
    return f(femb, xidx)


def _gather_body(xe, fedge, src, dst, ea0, ea1, pe_out, ge_out,
                 is0, id0, ie0, if0, is1, id1, ie1, if1,
                 ra0, rb0, rc0, rd0, ra1, rb1, rc1, rd1,
                 os1, os2, gsem0, gsem1, isem0, isem1, wsem):
    c = lax.axis_index("c")
    s = lax.axis_index("s")
    wid = c * NS + s
    iall = ((is0, id0, ie0, if0), (is1, id1, ie1, if1))
    srcs = (src, dst, ea0, ea1)
    bufs = ((ra0, rb0, rc0, rd0), (ra1, rb1, rc1, rd1))
    gsem = (gsem0, gsem1)
    isem = (isem0, isem1)

    def fire_idx(ci, p):
        base = wid * CE + ci * CH
        for k in range(4):
            pltpu.async_copy(srcs[k].at[pl.ds(base, CH)], iall[p][k],
                             isem[p])

    def fire_gathers(ci, p):
        ra, rb, rc, rd = bufs[p]
        for k in range(4):
            pltpu.make_async_copy(srcs[k].at[pl.ds(0, CH)], iall[p][k],
                                  isem[p]).wait()
        pltpu.async_copy(xe.at[iall[p][0]], ra, gsem[p])
        pltpu.async_copy(xe.at[iall[p][1]], rb, gsem[p])
        pltpu.async_copy(fedge.at[iall[p][2]], rc, gsem[p])
        pltpu.async_copy(fedge.at[iall[p][3]], rd, gsem[p])

    def step(ci, p, first, last):
        ra, rb, rc, rd = bufs[p]
        for buf in (ra, rb, rc, rd):
            pltpu.make_async_copy(xe.at[pl.ds(0, CH)], buf, gsem[p]).wait()
        if not last:
            fire_idx(ci + 2, p)
        if not first:
            pltpu.make_async_copy(pe_out.at[pl.ds(0, CH)], os1, wsem).wait()
            pltpu.make_async_copy(ge_out.at[pl.ds(0, CH)], os2, wsem).wait()

        def mul2(r, _):
            for cc in range(DIM // 16):
                sl = pl.ds(cc * 16, 16)
                os1[r, sl] = ra[r, sl] * rb[r, sl]
                os2[r, sl] = rc[r, sl] * rd[r, sl]
            return 0

        lax.fori_loop(0, CH, mul2, 0)
        base = wid * CE + ci * CH
        pltpu.async_copy(os1, pe_out.at[pl.ds(base, CH)], wsem)
        pltpu.async_copy(os2, ge_out.at[pl.ds(base, CH)], wsem)
        if not last:
            fire_gathers(ci + 2, p)

    fire_idx(0, 0)
    fire_gathers(0, 0)
    fire_idx(1, 1)
    fire_gathers(1, 1)
    step(0, 0, first=True, last=False)

    def loop(it, _):
        step(2 * it + 1, 1, first=False, last=False)
        step(2 * it + 2, 0, first=False, last=False)
        return 0

    lax.fori_loop(0, (NCHUNK - 3) // 2, loop, 0)
    step(NCHUNK - 2, 1, first=False, last=True)
    step(NCHUNK - 1, 0, first=False, last=True)
    pltpu.make_async_copy(pe_out.at[pl.ds(0, CH)], os1, wsem).wait()
    pltpu.make_async_copy(ge_out.at[pl.ds(0, CH)], os2, wsem).wait()


def _sc_gather(xe, fedge, src, dst, ea0, ea1):
    mesh = plsc.VectorSubcoreMesh(core_axis_name="c", subcore_axis_name="s",
                                  num_cores=NC, num_subcores=NS)
    f = pl.kernel(
        _gather_body,
        out_type=(jax.ShapeDtypeStruct((E2, DIM), jnp.float32),
                  jax.ShapeDtypeStruct((E2, DIM), jnp.float32)),
        mesh=mesh,
        scratch_types=(
            pltpu.VMEM((CH,), jnp.int32),
            pltpu.VMEM((CH,), jnp.int32),
            pltpu.VMEM((CH,), jnp.int32),
            pltpu.VMEM((CH,), jnp.int32),
            pltpu.VMEM((CH,), jnp.int32),
            pltpu.VMEM((CH,), jnp.int32),
            pltpu.VMEM((CH,), jnp.int32),
            pltpu.VMEM((CH,), jnp.int32),
            pltpu.VMEM((CH, DIM), jnp.float32),
            pltpu.VMEM((CH, DIM), jnp.float32),
            pltpu.VMEM((CH, DIM), jnp.float32),
            pltpu.VMEM((CH, DIM), jnp.float32),
            pltpu.VMEM((CH, DIM), jnp.float32),
            pltpu.VMEM((CH, DIM), jnp.float32),
            pltpu.VMEM((CH, DIM), jnp.float32),
            pltpu.VMEM((CH, DIM), jnp.float32),
            pltpu.VMEM((CH, DIM), jnp.float32),
            pltpu.VMEM((CH, DIM), jnp.float32),
            pltpu.SemaphoreType.DMA,
            pltpu.SemaphoreType.DMA,
            pltpu.SemaphoreType.DMA,
            pltpu.SemaphoreType.DMA,
            pltpu.SemaphoreType.DMA,
        ),
    )
    return f(xe, fedge, src, dst, ea0, ea1)


def _mlp_body(pe_ref, ge_ref, dst_ref, w1, b1, w2, b2, l1w, l1b, l2w, l2b,
              msg_ref, mdst_ref, l0_ref, ne_ref):
    i = pl.program_id(0)
    ge = ge_ref[...]
    h = jnp.maximum(
        lax.dot_general(ge, w1[...], (((1,), (1,)), ((), ())),
                        preferred_element_type=jnp.float32) + b1[...], 0.0)
    loc = jnp.sum(h * w2[...], axis=1, keepdims=True) + b2[...]
    sig = jax.nn.sigmoid(loc)
    sgate = jnp.clip(sig * (IMAX - IMIN) + IMIN, 0.0, 1.0)
    maskv = (sgate > 0.0).astype(jnp.float32)
    l0p = jnp.sum(jax.nn.sigmoid(loc - L0_SHIFT))
    nep = jnp.sum(maskv)

    pe = pe_ref[...]
    h2 = jnp.maximum(
        lax.dot_general(pe, l1w[...], (((1,), (1,)), ((), ())),
                        preferred_element_type=jnp.float32) + l1b[...], 0.0)
    pw = lax.dot_general(h2, l2w[...], (((1,), (1,)), ((), ())),
                         preferred_element_type=jnp.float32) + l2b[...]
    msg_ref[...] = pw * sgate

    dstv = dst_ref[0]
    mdst_ref[0] = jnp.where(sgate > 0.0, dstv, DUMP)

    @pl.when(i == 0)
    def _():
        l0_ref[...] = jnp.zeros((1, 1), jnp.float32)
        ne_ref[...] = jnp.zeros((1, 1), jnp.float32)

    l0_ref[...] += jnp.reshape(l0p, (1, 1))
    ne_ref[...] += jnp.reshape(nep, (1, 1))


def _tc_mlp(pe, ge, dst3, w1, b1, w2, b2, l1w, l1b, l2w, l2b):
    full = lambda shp: pl.BlockSpec(shp, lambda i: (0,) * len(shp))
    return pl.pallas_call(
        _mlp_body,
        grid=(NEB,),
        in_specs=[
            pl.BlockSpec((EB, DIM), lambda i: (i, 0)),
            pl.BlockSpec((EB, DIM), lambda i: (i, 0)),
            pl.BlockSpec((1, EB, 1), lambda i: (i, 0, 0)),
            full((H, DIM)), full((1, H)), full((1, H)), full((1, 1)),
            full((H, DIM)), full((1, H)), full((DIM, H)), full((1, DIM)),
        ],
        out_specs=[
            pl.BlockSpec((EB, DIM), lambda i: (i, 0)),
            pl.BlockSpec((1, EB, 1), lambda i: (i, 0, 0)),
            pl.BlockSpec((1, 1), lambda i: (0, 0)),
            pl.BlockSpec((1, 1), lambda i: (0, 0)),
        ],
        out_shape=[
            jax.ShapeDtypeStruct((E2, DIM), jnp.float32),
            jax.ShapeDtypeStruct((NEB, EB, 1), jnp.int32),
            jax.ShapeDtypeStruct((1, 1), jnp.float32),
            jax.ShapeDtypeStruct((1, 1), jnp.float32),
        ],
    )(pe, ge, dst3, w1, b1, w2, b2, l1w, l1b, l2w, l2b)


NPS = NP // NS
ZR = 128
SCH = 80


NSCH = CE // SCH


def _scatter_msg_body(msgx, dste, msum, mv0, mv1, dv0, dv1, zb,
                      lsem0, lsem1, ssem0, ssem1, acc):
    c = lax.axis_index("c")
    s = lax.axis_index("s")
    mv = (mv0, mv1)
    dv = (dv0, dv1)
    lsem = (lsem0, lsem1)
    ssem = (ssem0, ssem1)

    def zrow(r, _):
        for cc in range(DIM // 16):
            zb[r, pl.ds(cc * 16, 16)] = jnp.zeros((16,), jnp.float32)
        return 0

    lax.fori_loop(0, ZR, zrow, 0)
    stripe = s * NPS
    for k in range(NPS // ZR):
        pltpu.sync_copy(zb, acc.at[pl.ds(stripe + k * ZR, ZR)])
    plsc.subcore_barrier()

    ebase = c * (E2 // NC) + s * CE

    def fire_loads(ci, p):
        pltpu.async_copy(dste.at[pl.ds(ebase + ci * SCH, SCH)], dv[p],
                         lsem[p])
        pltpu.async_copy(msgx.at[pl.ds(ebase + ci * SCH, SCH)], mv[p],
                         lsem[p])

    def step(ci, p, first, last):
        q = 1 - p
        pltpu.make_async_copy(dste.at[pl.ds(0, SCH)], dv[p], lsem[p]).wait()
        pltpu.make_async_copy(msgx.at[pl.ds(0, SCH)], mv[p], lsem[p]).wait()
        pltpu.async_copy(mv[p], acc.at[dv[p]], ssem[p], add=True)
        if not first:
            pltpu.make_async_copy(msgx.at[pl.ds(0, SCH)], mv[q],
                                  ssem[q]).wait()
        if not last:
            fire_loads(ci + 1, q)

    fire_loads(0, 0)
    step(0, 0, first=True, last=False)

    def loop(it, _):
        step(2 * it + 1, 1, first=False, last=False)
        step(2 * it + 2, 0, first=False, last=False)
        return 0

    lax.fori_loop(0, (NSCH - 3) // 2, loop, 0)
    step(NSCH - 2, 1, first=False, last=False)
    step(NSCH - 1, 0, first=False, last=True)
    pltpu.make_async_copy(msgx.at[pl.ds(0, SCH)], mv0, ssem0).wait()
    plsc.subcore_barrier()

    for k in range(NPS // ZR):
        sl = pl.ds(stripe + k * ZR, ZR)
        pltpu.sync_copy(acc.at[sl], zb)
        pltpu.sync_copy(zb, msum.at[pl.ds(c * NP + stripe + k * ZR, ZR)])


def _sc_scatter_msg(msgx, dste):
    mesh = plsc.VectorSubcoreMesh(core_axis_name="c", subcore_axis_name="s",
                                  num_cores=NC, num_subcores=NS)
    f = pl.kernel(
        _scatter_msg_body,
        out_type=jax.ShapeDtypeStruct((NC * NP, DIM), jnp.float32),
        mesh=mesh,
        scratch_types=(
            pltpu.VMEM((SCH, DIM), jnp.float32),
            pltpu.VMEM((SCH, DIM), jnp.float32),
            pltpu.VMEM((SCH,), jnp.int32),
            pltpu.VMEM((SCH,), jnp.int32),
            pltpu.VMEM((ZR, DIM), jnp.float32),
            pltpu.SemaphoreType.DMA,
            pltpu.SemaphoreType.DMA,
            pltpu.SemaphoreType.DMA,
            pltpu.SemaphoreType.DMA,
            pltpu.VMEM_SHARED((NP, DIM), jnp.float32),
        ),
    )
    return f(msgx, dste)


def _scatter_cnt_body(mdst, csum, ones, dv0, dv1, zb,
                      lsem0, lsem1, ssem0, ssem1, acc):
    c = lax.axis_index("c")
    s = lax.axis_index("s")
    dv = (dv0, dv1)
    lsem = (lsem0, lsem1)
    ssem = (ssem0, ssem1)

    def zrow(r, _):
        for cc in range(DIM // 16):
            zb[r, pl.ds(cc * 16, 16)] = jnp.zeros((16,), jnp.float32)
        return 0

    lax.fori_loop(0, ZR, zrow, 0)

    e0 = jnp.where(lax.iota(jnp.int32, 16) == 0,
                   jnp.float32(1.0), jnp.float32(0.0))

    def orow(r, _):
        ones[r, pl.ds(0, 16)] = e0
        for cc in range(1, DIM // 16):
            ones[r, pl.ds(cc * 16, 16)] = jnp.zeros((16,), jnp.float32)
        return 0

    lax.fori_loop(0, SCH, orow, 0)
    stripe = s * NPS
    for k in range(NPS // ZR):
        pltpu.sync_copy(zb, acc.at[pl.ds(stripe + k * ZR, ZR)])
    plsc.subcore_barrier()

    ebase = c * (E2 // NC) + s * CE

    def fire_load(ci, p):
        pltpu.async_copy(mdst.at[pl.ds(ebase + ci * SCH, SCH)], dv[p],
                         lsem[p])

    def step(ci, p, first, last):
        q = 1 - p
        pltpu.make_async_copy(mdst.at[pl.ds(0, SCH)], dv[p], lsem[p]).wait()
        pltpu.async_copy(ones, acc.at[dv[p]], ssem[p], add=True)
        if not first:
            pltpu.make_async_copy(csum.at[pl.ds(0, SCH)], ones,
                                  ssem[q]).wait()
        if not last:
            fire_load(ci + 1, q)

    fire_load(0, 0)
    step(0, 0, first=True, last=False)

    def loop(it, _):
        step(2 * it + 1, 1, first=False, last=False)
        step(2 * it + 2, 0, first=False, last=False)
        return 0

    lax.fori_loop(0, (NSCH - 3) // 2, loop, 0)
    step(NSCH - 2, 1, first=False, last=False)
    step(NSCH - 1, 0, first=False, last=True)
    pltpu.make_async_copy(csum.at[pl.ds(0, SCH)], ones, ssem0).wait()
    plsc.subcore_barrier()

    for k in range(NPS // ZR):
        sl = pl.ds(stripe + k * ZR, ZR)
        pltpu.sync_copy(acc.at[sl], zb)
        pltpu.sync_copy(zb, csum.at[pl.ds(c * NP + stripe + k * ZR, ZR)])


def _sc_scatter_cnt(mdst):
    mesh = plsc.VectorSubcoreMesh(core_axis_name="c", subcore_axis_name="s",
                                  num_cores=NC, num_subcores=NS)
    f = pl.kernel(
        _scatter_cnt_body,
        out_type=jax.ShapeDtypeStruct((NC * NP, DIM), jnp.float32),
        mesh=mesh,
        scratch_types=(
            pltpu.VMEM((SCH, DIM), jnp.float32),
            pltpu.VMEM((SCH,), jnp.int32),
            pltpu.VMEM((SCH,), jnp.int32),
            pltpu.VMEM((ZR, DIM), jnp.float32),
            pltpu.SemaphoreType.DMA,
            pltpu.SemaphoreType.DMA,
            pltpu.SemaphoreType.DMA,
            pltpu.SemaphoreType.DMA,
            pltpu.VMEM_SHARED((NP, DIM), jnp.float32),
        ),
    )
    return f(mdst)


def _fin_body(msum_ref, csum_ref, batch_ref, gw, gb,
              out_ref, l2_ref, pooled, gcnt):
    i = pl.program_id(0)

    @pl.when(i == 0)
    def _():
        pooled[...] = jnp.zeros((G, DIM), jnp.float32)
        gcnt[...] = jnp.zeros((G, 1), jnp.float32)
        l2_ref[...] = jnp.zeros((1, 1), jnp.float32)

    mm = msum_ref[...]
    m = mm[0] + mm[1]
    cc = csum_ref[...]
    cn = cc[0, :, 0:1] + cc[1, :, 0:1]
    upd = m / jnp.maximum(cn, 1.0)
    l2_ref[...] += jnp.reshape(jnp.sum(upd * upd), (1, 1))

    bv = batch_ref[0, 0, :]
    oh = (bv[None, :] == lax.broadcasted_iota(jnp.int32, (G, NB), 0)
          ).astype(jnp.float32)
    pooled[...] += lax.dot_general(oh, upd, (((1,), (0,)), ((), ())),
                                   preferred_element_type=jnp.float32)
    gcnt[...] += jnp.sum(oh, axis=1, keepdims=True)

    @pl.when(i == NBLK - 1)
    def _():
        p = pooled[...] / jnp.maximum(gcnt[...], 1.0)
        out_ref[...] = lax.dot_general(p, gw[...], (((1,), (1,)), ((), ())),
                                       preferred_element_type=jnp.float32) \
            + gb[...]


def _tc_finish(msum, csum, batch3, gw, gb):
    return pl.pallas_call(
        _fin_body,
        grid=(NBLK,),
        in_specs=[
            pl.BlockSpec((2, NB, DIM), lambda i: (0, i, 0)),
            pl.BlockSpec((2, NB, DIM), lambda i: (0, i, 0)),
            pl.BlockSpec((1, 1, NB), lambda i: (i, 0, 0)),
            pl.BlockSpec((2, DIM), lambda i: (0, 0)),
            pl.BlockSpec((1, 2), lambda i: (0, 0)),
        ],
        out_specs=[
            pl.BlockSpec((G, 2), lambda i: (0, 0)),
            pl.BlockSpec((1, 1), lambda i: (0, 0)),
        ],
        out_shape=[
            jax.ShapeDtypeStruct((G, 2), jnp.float32),
            jax.ShapeDtypeStruct((1, 1), jnp.float32),
        ],
        scratch_shapes=[
            pltpu.VMEM((G, DIM), jnp.float32),
            pltpu.VMEM((G, 1), jnp.float32),
        ],
    )(msum, csum, batch3, gw, gb)


def kernel(x, edge_index, edge_attr, batch, is_training,
           feature_emb, fe_edge, lp_w1, lp_b1, lp_w2, lp_b2,
           lin1_w, lin1_b, lin2_w, lin2_b, g_w, g_b):
    xidx = x[:, 0].astype(jnp.int32)
    src = edge_index[0].astype(jnp.int32)
    dst = edge_index[1].astype(jnp.int32)
    ea0 = edge_attr[:, 0].astype(jnp.int32)
    ea1 = edge_attr[:, 1].astype(jnp.int32)

    xe = _sc_xe(feature_emb, xidx)

    pe, ge = _sc_gather(xe, fe_edge, src, dst, ea0, ea1)
    msg, mdst3, l0_sum, ne_sum = _tc_mlp(
        pe, ge, dst.reshape(NEB, EB, 1),
        lp_w1, lp_b1.reshape(1, H), lp_w2.reshape(1, H), lp_b2.reshape(1, 1),
        lin1_w, lin1_b.reshape(1, H), lin2_w, lin2_b.reshape(1, DIM))
    msum = _sc_scatter_msg(msg, dst).reshape(NC, NP, DIM)
    csum = _sc_scatter_cnt(mdst3.reshape(E2)).reshape(NC, NP, DIM)

    batch_pad = jnp.concatenate(
        [batch.astype(jnp.int32), jnp.full((NP - N,), G, jnp.int32)])
    out, l2 = _tc_finish(msum, csum, batch_pad.reshape(NBLK, 1, NB),
                         g_w, g_b.reshape(1, 2))

    l0 = l0_sum[0, 0] / float(E)
    l2s = l2[0, 0]
    num_edges = ne_sum[0, 0].astype(jnp.int32)
    return out, l0, l2s, num_edges


# speedup vs baseline: 1.3370x; 1.0572x over previous
import jax
import jax.numpy as jnp
import numpy as np
from jax import lax
from jax.experimental import pallas as pl
from jax.experimental.pallas import tpu as pltpu
from jax.experimental.pallas import tpu_sc as plsc

N = 10000
E = 320000
DIM = 128
H = 256
G = 128
TEMP = 0.66
IMIN = -0.1
IMAX = 1.1
L0_SHIFT = float(TEMP * np.log2(-IMIN / IMAX))

NC = 2
NS = 16
NW = NC * NS
NH = 1
E2 = E // NH
CE = E2 // NW
CH = 80
NCHUNK = CE // CH

EB = 4000
NEB = E2 // EB

NP = 10240
DUMP = N
NB = 640
NBLK = NP // NB


XCH = 80
NXCH = N // XCH


def _xe_body(femb, xidx, xe_out, iv, rv, sem):
    c = lax.axis_index("c")
    s = lax.axis_index("s")
    wid = c * NS + s

    for k in range((NXCH + NW - 1) // NW):
        cid = wid + k * NW

        @pl.when(cid < NXCH)
        def _():
            base = cid * XCH
            pltpu.sync_copy(xidx.at[pl.ds(base, XCH)], iv)
            pltpu.async_copy(femb.at[iv], rv, sem).wait()
            pltpu.sync_copy(rv, xe_out.at[pl.ds(base, XCH)])


def _sc_xe(femb, xidx):
    mesh = plsc.VectorSubcoreMesh(core_axis_name="c", subcore_axis_name="s",
                                  num_cores=NC, num_subcores=NS)
    f = pl.kernel(
        _xe_body,
        out_type=jax.ShapeDtypeStruct((N, DIM), jnp.float32),
        mesh=mesh,
        scratch_types=(
            pltpu.VMEM((XCH,), jnp.int32),
            pltpu.VMEM((XCH, DIM), jnp.float32),
            pltpu.SemaphoreType.DMA,
        ),
    )
    return f(femb, xidx)


def _gather_body(xe, fedge, src, dst, ea0, ea1, pe_out, ge_out,
                 is0, id0, ie0, if0, is1, id1, ie1, if1,
                 ra0, rb0, rc0, rd0, ra1, rb1, rc1, rd1,
                 os1, os2, gsem0, gsem1, isem0, isem1, wsem):
    c = lax.axis_index("c")
    s = lax.axis_index("s")
    wid = c * NS + s
    iall = ((is0, id0, ie0, if0), (is1, id1, ie1, if1))
    srcs = (src, dst, ea0, ea1)
    bufs = ((ra0, rb0, rc0, rd0), (ra1, rb1, rc1, rd1))
    gsem = (gsem0, gsem1)
    isem = (isem0, isem1)

    def fire_idx(ci, p):
        base = wid * CE + ci * CH
        for k in range(4):
            pltpu.async_copy(srcs[k].at[pl.ds(base, CH)], iall[p][k],
                             isem[p])

    def fire_gathers(ci, p):
        ra, rb, rc, rd = bufs[p]
        for k in range(4):
            pltpu.make_async_copy(srcs[k].at[pl.ds(0, CH)], iall[p][k],
                                  isem[p]).wait()
        pltpu.async_copy(xe.at[iall[p][0]], ra, gsem[p])
        pltpu.async_copy(xe.at[iall[p][1]], rb, gsem[p])
        pltpu.async_copy(fedge.at[iall[p][2]], rc, gsem[p])
        pltpu.async_copy(fedge.at[iall[p][3]], rd, gsem[p])

    def step(ci, p, first, last):
        ra, rb, rc, rd = bufs[p]
        for buf in (ra, rb, rc, rd):
            pltpu.make_async_copy(xe.at[pl.ds(0, CH)], buf, gsem[p]).wait()
        if not last:
            fire_idx(ci + 2, p)
        if not first:
            pltpu.make_async_copy(pe_out.at[pl.ds(0, CH)], os1, wsem).wait()
            pltpu.make_async_copy(ge_out.at[pl.ds(0, CH)], os2, wsem).wait()

        def mul2(r, _):
            for cc in range(DIM // 16):
                sl = pl.ds(cc * 16, 16)
                os1[r, sl] = ra[r, sl] * rb[r, sl]
                os2[r, sl] = rc[r, sl] * rd[r, sl]
            return 0

        lax.fori_loop(0, CH, mul2, 0)
        base = wid * CE + ci * CH
        pltpu.async_copy(os1, pe_out.at[pl.ds(base, CH)], wsem)
        pltpu.async_copy(os2, ge_out.at[pl.ds(base, CH)], wsem)
        if not last:
            fire_gathers(ci + 2, p)

    fire_idx(0, 0)
    fire_gathers(0, 0)
    fire_idx(1, 1)
    fire_gathers(1, 1)
    step(0, 0, first=True, last=False)

    def loop(it, _):
        step(2 * it + 1, 1, first=False, last=False)
        step(2 * it + 2, 0, first=False, last=False)
        return 0

    lax.fori_loop(0, (NCHUNK - 3) // 2, loop, 0)
    step(NCHUNK - 2, 1, first=False, last=True)
    step(NCHUNK - 1, 0, first=False, last=True)
    pltpu.make_async_copy(pe_out.at[pl.ds(0, CH)], os1, wsem).wait()
    pltpu.make_async_copy(ge_out.at[pl.ds(0, CH)], os2, wsem).wait()


def _sc_gather(xe, fedge, src, dst, ea0, ea1):
    mesh = plsc.VectorSubcoreMesh(core_axis_name="c", subcore_axis_name="s",
                                  num_cores=NC, num_subcores=NS)
    f = pl.kernel(
        _gather_body,
        out_type=(jax.ShapeDtypeStruct((E2, DIM), jnp.float32),
                  jax.ShapeDtypeStruct((E2, DIM), jnp.float32)),
        mesh=mesh,
        scratch_types=(
            pltpu.VMEM((CH,), jnp.int32),
            pltpu.VMEM((CH,), jnp.int32),
            pltpu.VMEM((CH,), jnp.int32),
            pltpu.VMEM((CH,), jnp.int32),
            pltpu.VMEM((CH,), jnp.int32),
            pltpu.VMEM((CH,), jnp.int32),
            pltpu.VMEM((CH,), jnp.int32),
            pltpu.VMEM((CH,), jnp.int32),
            pltpu.VMEM((CH, DIM), jnp.float32),
            pltpu.VMEM((CH, DIM), jnp.float32),
            pltpu.VMEM((CH, DIM), jnp.float32),
            pltpu.VMEM((CH, DIM), jnp.float32),
            pltpu.VMEM((CH, DIM), jnp.float32),
            pltpu.VMEM((CH, DIM), jnp.float32),
            pltpu.VMEM((CH, DIM), jnp.float32),
            pltpu.VMEM((CH, DIM), jnp.float32),
            pltpu.VMEM((CH, DIM), jnp.float32),
            pltpu.VMEM((CH, DIM), jnp.float32),
            pltpu.SemaphoreType.DMA,
            pltpu.SemaphoreType.DMA,
            pltpu.SemaphoreType.DMA,
            pltpu.SemaphoreType.DMA,
            pltpu.SemaphoreType.DMA,
        ),
    )
    return f(xe, fedge, src, dst, ea0, ea1)


def _mlp_body(pe_ref, ge_ref, dst_ref, w1, b1, w2, b2, l1w, l1b, l2w, l2b,
              msg_ref, mdst_ref, l0_ref, ne_ref):
    i = pl.program_id(0)
    ge = ge_ref[...]
    h = jnp.maximum(
        lax.dot_general(ge, w1[...], (((1,), (1,)), ((), ())),
                        preferred_element_type=jnp.float32) + b1[...], 0.0)
    loc = jnp.sum(h * w2[...], axis=1, keepdims=True) + b2[...]
    sig = jax.nn.sigmoid(loc)
    sgate = jnp.clip(sig * (IMAX - IMIN) + IMIN, 0.0, 1.0)
    maskv = (sgate > 0.0).astype(jnp.float32)
    l0p = jnp.sum(jax.nn.sigmoid(loc - L0_SHIFT))
    nep = jnp.sum(maskv)

    pe = pe_ref[...]
    h2 = jnp.maximum(
        lax.dot_general(pe, l1w[...], (((1,), (1,)), ((), ())),
                        preferred_element_type=jnp.float32) + l1b[...], 0.0)
    pw = lax.dot_general(h2, l2w[...], (((1,), (1,)), ((), ())),
                         preferred_element_type=jnp.float32) + l2b[...]
    msg_ref[...] = pw * sgate

    dstv = dst_ref[0]
    mdst_ref[0] = jnp.where(sgate > 0.0, dstv, DUMP)

    @pl.when(i == 0)
    def _():
        l0_ref[...] = jnp.zeros((1, 1), jnp.float32)
        ne_ref[...] = jnp.zeros((1, 1), jnp.float32)

    l0_ref[...] += jnp.reshape(l0p, (1, 1))
    ne_ref[...] += jnp.reshape(nep, (1, 1))


def _tc_mlp(pe, ge, dst3, w1, b1, w2, b2, l1w, l1b, l2w, l2b):
    full = lambda shp: pl.BlockSpec(shp, lambda i: (0,) * len(shp))
    return pl.pallas_call(
        _mlp_body,
        grid=(NEB,),
        in_specs=[
            pl.BlockSpec((EB, DIM), lambda i: (i, 0)),
            pl.BlockSpec((EB, DIM), lambda i: (i, 0)),
            pl.BlockSpec((1, EB, 1), lambda i: (i, 0, 0)),
            full((H, DIM)), full((1, H)), full((1, H)), full((1, 1)),
            full((H, DIM)), full((1, H)), full((DIM, H)), full((1, DIM)),
        ],
        out_specs=[
            pl.BlockSpec((EB, DIM), lambda i: (i, 0)),
            pl.BlockSpec((1, EB, 1), lambda i: (i, 0, 0)),
            pl.BlockSpec((1, 1), lambda i: (0, 0)),
            pl.BlockSpec((1, 1), lambda i: (0, 0)),
        ],
        out_shape=[
            jax.ShapeDtypeStruct((E2, DIM), jnp.float32),
            jax.ShapeDtypeStruct((NEB, EB, 1), jnp.int32),
            jax.ShapeDtypeStruct((1, 1), jnp.float32),
            jax.ShapeDtypeStruct((1, 1), jnp.float32),
        ],
    )(pe, ge, dst3, w1, b1, w2, b2, l1w, l1b, l2w, l2b)


NPS = NP // NS
ZR = 128
SCH = 80


NSCH = CE // SCH


def _scatter_msg_body(msgx, dste, msum, mv0, mv1, dv0, dv1, zb,
                      lsem0, lsem1, ssem0, ssem1, acc):
    c = lax.axis_index("c")
    s = lax.axis_index("s")
    mv = (mv0, mv1)
    dv = (dv0, dv1)
    lsem = (lsem0, lsem1)
    ssem = (ssem0, ssem1)

    def zrow(r, _):
        for cc in range(DIM // 16):
            zb[r, pl.ds(cc * 16, 16)] = jnp.zeros((16,), jnp.float32)
        return 0

    lax.fori_loop(0, ZR, zrow, 0)
    stripe = s * NPS
    for k in range(NPS // ZR):
        pltpu.sync_copy(zb, acc.at[pl.ds(stripe + k * ZR, ZR)])
    plsc.subcore_barrier()

    ebase = c * (E2 // NC) + s * CE

    def fire_loads(ci, p):
        pltpu.async_copy(dste.at[pl.ds(ebase + ci * SCH, SCH)], dv[p],
                         lsem[p])
        pltpu.async_copy(msgx.at[pl.ds(ebase + ci * SCH, SCH)], mv[p],
                         lsem[p])

    def step(ci, p, first, last):
        q = 1 - p
        pltpu.make_async_copy(dste.at[pl.ds(0, SCH)], dv[p], lsem[p]).wait()
        pltpu.make_async_copy(msgx.at[pl.ds(0, SCH)], mv[p], lsem[p]).wait()
        pltpu.async_copy(mv[p], acc.at[dv[p]], ssem[p], add=True)
        if not first:
            pltpu.make_async_copy(msgx.at[pl.ds(0, SCH)], mv[q],
                                  ssem[q]).wait()
        if not last:
            fire_loads(ci + 1, q)

    fire_loads(0, 0)
    step(0, 0, first=True, last=False)

    def loop(it, _):
        step(2 * it + 1, 1, first=False, last=False)
        step(2 * it + 2, 0, first=False, last=False)
        return 0

    lax.fori_loop(0, (NSCH - 3) // 2, loop, 0)
    step(NSCH - 2, 1, first=False, last=False)
    step(NSCH - 1, 0, first=False, last=True)
    pltpu.make_async_copy(msgx.at[pl.ds(0, SCH)], mv0, ssem0).wait()
    plsc.subcore_barrier()

    for k in range(NPS // ZR):
        sl = pl.ds(stripe + k * ZR, ZR)
        pltpu.sync_copy(acc.at[sl], zb)
        pltpu.sync_copy(zb, msum.at[pl.ds(c * NP + stripe + k * ZR, ZR)])


def _sc_scatter_msg(msgx, dste):
    mesh = plsc.VectorSubcoreMesh(core_axis_name="c", subcore_axis_name="s",
                                  num_cores=NC, num_subcores=NS)
    f = pl.kernel(
        _scatter_msg_body,
        out_type=jax.ShapeDtypeStruct((NC * NP, DIM), jnp.float32),
        mesh=mesh,
        scratch_types=(
            pltpu.VMEM((SCH, DIM), jnp.float32),
            pltpu.VMEM((SCH, DIM), jnp.float32),
            pltpu.VMEM((SCH,), jnp.int32),
            pltpu.VMEM((SCH,), jnp.int32),
            pltpu.VMEM((ZR, DIM), jnp.float32),
            pltpu.SemaphoreType.DMA,
            pltpu.SemaphoreType.DMA,
            pltpu.SemaphoreType.DMA,
            pltpu.SemaphoreType.DMA,
            pltpu.VMEM_SHARED((NP, DIM), jnp.float32),
        ),
    )
    return f(msgx, dste)


def _scatter_cnt_body(mdst, csum, ones, dv0, dv1, zb,
                      lsem0, lsem1, ssem0, ssem1, acc):
    c = lax.axis_index("c")
    s = lax.axis_index("s")
    dv = (dv0, dv1)
    lsem = (lsem0, lsem1)
    ssem = (ssem0, ssem1)

    def zrow(r, _):
        for cc in range(DIM // 16):
            zb[r, pl.ds(cc * 16, 16)] = jnp.zeros((16,), jnp.float32)
        return 0

    lax.fori_loop(0, ZR, zrow, 0)

    e0 = jnp.where(lax.iota(jnp.int32, 16) == 0,
                   jnp.float32(1.0), jnp.float32(0.0))

    def orow(r, _):
        ones[r, pl.ds(0, 16)] = e0
        for cc in range(1, DIM // 16):
            ones[r, pl.ds(cc * 16, 16)] = jnp.zeros((16,), jnp.float32)
        return 0

    lax.fori_loop(0, SCH, orow, 0)
    stripe = s * NPS
    for k in range(NPS // ZR):
        pltpu.sync_copy(zb, acc.at[pl.ds(stripe + k * ZR, ZR)])
    plsc.subcore_barrier()

    ebase = c * (E2 // NC) + s * CE

    def fire_load(ci, p):
        pltpu.async_copy(mdst.at[pl.ds(ebase + ci * SCH, SCH)], dv[p],
                         lsem[p])

    def step(ci, p, first, last):
        q = 1 - p
        pltpu.make_async_copy(mdst.at[pl.ds(0, SCH)], dv[p], lsem[p]).wait()
        pltpu.async_copy(ones, acc.at[dv[p]], ssem[p], add=True)
        if not first:
            pltpu.make_async_copy(csum.at[pl.ds(0, SCH)], ones,
                                  ssem[q]).wait()
        if not last:
            fire_load(ci + 1, q)

    fire_load(0, 0)
    step(0, 0, first=True, last=False)

    def loop(it, _):
        step(2 * it + 1, 1, first=False, last=False)
        step(2 * it + 2, 0, first=False, last=False)
        return 0

    lax.fori_loop(0, (NSCH - 3) // 2, loop, 0)
    step(NSCH - 2, 1, first=False, last=False)
    step(NSCH - 1, 0, first=False, last=True)
    pltpu.make_async_copy(csum.at[pl.ds(0, SCH)], ones, ssem0).wait()
    plsc.subcore_barrier()

    for k in range(NPS // ZR):
        sl = pl.ds(stripe + k * ZR, ZR)
        pltpu.sync_copy(acc.at[sl], zb)
        pltpu.sync_copy(zb, csum.at[pl.ds(c * NP + stripe + k * ZR, ZR)])


def _sc_scatter_cnt(mdst):
    mesh = plsc.VectorSubcoreMesh(core_axis_name="c", subcore_axis_name="s",
                                  num_cores=NC, num_subcores=NS)
    f = pl.kernel(
        _scatter_cnt_body,
        out_type=jax.ShapeDtypeStruct((NC * NP, DIM), jnp.float32),
        mesh=mesh,
        scratch_types=(
            pltpu.VMEM((SCH, DIM), jnp.float32),
            pltpu.VMEM((SCH,), jnp.int32),
            pltpu.VMEM((SCH,), jnp.int32),
            pltpu.VMEM((ZR, DIM), jnp.float32),
            pltpu.SemaphoreType.DMA,
            pltpu.SemaphoreType.DMA,
            pltpu.SemaphoreType.DMA,
            pltpu.SemaphoreType.DMA,
            pltpu.VMEM_SHARED((NP, DIM), jnp.float32),
        ),
    )
    return f(mdst)


def _fin_body(msum_ref, csum_ref, batch_ref, gw, gb,
              out_ref, l2_ref, pooled, gcnt):
    i = pl.program_id(0)

    @pl.when(i == 0)
    def _():
        pooled[...] = jnp.zeros((G, DIM), jnp.float32)
        gcnt[...] = jnp.zeros((G, 1), jnp.float32)
        l2_ref[...] = jnp.zeros((1, 1), jnp.float32)

    mm = msum_ref[...]
    m = mm[0] + mm[1]
    cc = csum_ref[...]
    cn = cc[0, :, 0:1] + cc[1, :, 0:1]
    upd = m / jnp.maximum(cn, 1.0)
    l2_ref[...] += jnp.reshape(jnp.sum(upd * upd), (1, 1))

    bv = batch_ref[0, 0, :]
    oh = (bv[None, :] == lax.broadcasted_iota(jnp.int32, (G, NB), 0)
          ).astype(jnp.float32)
    pooled[...] += lax.dot_general(oh, upd, (((1,), (0,)), ((), ())),
                                   preferred_element_type=jnp.float32)
    gcnt[...] += jnp.sum(oh, axis=1, keepdims=True)

    @pl.when(i == NBLK - 1)
    def _():
        p = pooled[...] / jnp.maximum(gcnt[...], 1.0)
        out_ref[...] = lax.dot_general(p, gw[...], (((1,), (1,)), ((), ())),
                                       preferred_element_type=jnp.float32) \
            + gb[...]


def _tc_finish(msum, csum, batch3, gw, gb):
    return pl.pallas_call(
        _fin_body,
        grid=(NBLK,),
        in_specs=[
            pl.BlockSpec((2, NB, DIM), lambda i: (0, i, 0)),
            pl.BlockSpec((2, NB, DIM), lambda i: (0, i, 0)),
            pl.BlockSpec((1, 1, NB), lambda i: (i, 0, 0)),
            pl.BlockSpec((2, DIM), lambda i: (0, 0)),
            pl.BlockSpec((1, 2), lambda i: (0, 0)),
        ],
        out_specs=[
            pl.BlockSpec((G, 2), lambda i: (0, 0)),
            pl.BlockSpec((1, 1), lambda i: (0, 0)),
        ],
        out_shape=[
            jax.ShapeDtypeStruct((G, 2), jnp.float32),
            jax.ShapeDtypeStruct((1, 1), jnp.float32),
        ],
        scratch_shapes=[
            pltpu.VMEM((G, DIM), jnp.float32),
            pltpu.VMEM((G, 1), jnp.float32),
        ],
    )(msum, csum, batch3, gw, gb)


def kernel(x, edge_index, edge_attr, batch, is_training,
           feature_emb, fe_edge, lp_w1, lp_b1, lp_w2, lp_b2,
           lin1_w, lin1_b, lin2_w, lin2_b, g_w, g_b):
    xidx = x[:, 0].astype(jnp.int32)
    src = edge_index[0].astype(jnp.int32)
    dst = edge_index[1].astype(jnp.int32)
    ea0 = edge_attr[:, 0].astype(jnp.int32)
    ea1 = edge_attr[:, 1].astype(jnp.int32)

    xe = _sc_xe(feature_emb, xidx)

    pe, ge = _sc_gather(xe, fe_edge, src, dst, ea0, ea1)
    msg, mdst3, l0_sum, ne_sum = _tc_mlp(
        pe, ge, dst.reshape(NEB, EB, 1),
        lp_w1, lp_b1.reshape(1, H), lp_w2.reshape(1, H), lp_b2.reshape(1, 1),
        lin1_w, lin1_b.reshape(1, H), lin2_w, lin2_b.reshape(1, DIM))
    msum = _sc_scatter_msg(msg, dst).reshape(NC, NP, DIM)
    csum = _sc_scatter_cnt(mdst3.reshape(E2)).reshape(NC, NP, DIM)

    batch_pad = jnp.concatenate(
        [batch.astype(jnp.int32), jnp.full((NP - N,), G, jnp.int32)])
    out, l2 = _tc_finish(msum, csum, batch_pad.reshape(NBLK, 1, NB),
                         g_w, g_b.reshape(1, 2))

    l0 = l0_sum[0, 0] / float(E)
    l2s = l2[0, 0]
    num_edges = ne_sum[0, 0].astype(jnp.int32)
    return out, l0, l2s, num_edges
